# baseline (speedup 1.0000x reference)
import jax
import jax.numpy as jnp
from jax.experimental import pallas as pl
from jax.experimental.pallas import tpu as pltpu

_MEMORY_SIZE = 65536
_KEY_DIM = 64
_VALUE_DIM = 64
_BQ = 1024
_BM = 2048
_NUM_M_BLOCKS = _MEMORY_SIZE // _BM
_SCALE = 1.4426950408889634 / (_KEY_DIM ** 0.5)


def _attn_kernel(q_ref, k_ref, va_ref, wq_ref, bq_ref, wv_ref, bv_ref,
                 o_ref, q_scratch, acc_scratch, m_scratch):
    i = pl.program_id(0)

    @pl.when(i == 0)
    def _init():
        q = jax.lax.dot_general(q_ref[...], wq_ref[...],
                                (((1,), (0,)), ((), ())),
                                preferred_element_type=jnp.float32)
        q_scratch[...] = ((q + bq_ref[...]) * _SCALE).astype(jnp.bfloat16)
        m_scratch[...] = jnp.full_like(m_scratch, -jnp.inf)
        acc_scratch[...] = jnp.zeros_like(acc_scratch)

    s = jax.lax.dot_general(q_scratch[...], k_ref[...],
                            (((1,), (1,)), ((), ())),
                            preferred_element_type=jnp.float32)
    m_prev = m_scratch[...]
    m_cur = jnp.max(s, axis=1, keepdims=True)
    m_next = jnp.maximum(m_prev, m_cur)
    alpha = jnp.exp2(m_prev - m_next)
    p_b = jnp.exp2(s - m_next[:, :1]).astype(jnp.bfloat16)
    pv = jax.lax.dot_general(p_b, va_ref[...], (((1,), (0,)), ((), ())),
                             preferred_element_type=jnp.float32)
    acc_scratch[...] = acc_scratch[...] * alpha[:, :1] + pv
    m_scratch[...] = m_next

    @pl.when(i == _NUM_M_BLOCKS - 1)
    def _fin():
        read = acc_scratch[:, :_VALUE_DIM] / acc_scratch[:, _VALUE_DIM:_VALUE_DIM + 1]
        out = jax.lax.dot_general(read, wv_ref[...], (((1,), (0,)), ((), ())),
                                  preferred_element_type=jnp.float32)
        o_ref[...] = out + bv_ref[...]


def _attention(q2d, mem_keys_b, v_aug, Wq, bq2, Wv, bv2, interpret=False):
    return pl.pallas_call(
        _attn_kernel,
        grid=(_NUM_M_BLOCKS,),
        in_specs=[
            pl.BlockSpec((_BQ, _KEY_DIM), lambda i: (0, 0)),
            pl.BlockSpec((_BM, _KEY_DIM), lambda i: (i, 0)),
            pl.BlockSpec((_BM, 128), lambda i: (i, 0)),
            pl.BlockSpec((_KEY_DIM, _KEY_DIM), lambda i: (0, 0)),
            pl.BlockSpec((1, _KEY_DIM), lambda i: (0, 0)),
            pl.BlockSpec((_VALUE_DIM, _VALUE_DIM), lambda i: (0, 0)),
            pl.BlockSpec((1, _VALUE_DIM), lambda i: (0, 0)),
        ],
        out_specs=pl.BlockSpec((_BQ, _VALUE_DIM), lambda i: (0, 0)),
        out_shape=jax.ShapeDtypeStruct((_BQ, _VALUE_DIM), jnp.float32),
        scratch_shapes=[
            pltpu.VMEM((_BQ, _KEY_DIM), jnp.bfloat16),
            pltpu.VMEM((_BQ, 128), jnp.float32),
            pltpu.VMEM((_BQ, 128), jnp.float32),
        ],
        compiler_params=pltpu.CompilerParams(
            dimension_semantics=("arbitrary",)),
        interpret=interpret,
    )(q2d, mem_keys_b, v_aug, Wq, bq2, Wv, bv2)


def kernel(queries, mem_keys, mem_values, Wq, bq, Wv, bv):
    b, n, _ = queries.shape
    q2d = queries.reshape(b * n, _KEY_DIM)
    k_b = mem_keys.astype(jnp.bfloat16)
    v_aug = jnp.concatenate(
        [mem_values.astype(jnp.bfloat16),
         jnp.ones((_MEMORY_SIZE, 1), dtype=jnp.bfloat16),
         jnp.zeros((_MEMORY_SIZE, 128 - _VALUE_DIM - 1), dtype=jnp.bfloat16)],
        axis=1)
    out = _attention(q2d, k_b, v_aug,
                     Wq, bq.reshape(1, -1), Wv, bv.reshape(1, -1))
    return out.reshape(b, n, _VALUE_DIM)


# speedup vs baseline: 1.4401x; 1.4401x over previous
import jax
import jax.numpy as jnp
from jax.experimental import pallas as pl
from jax.experimental.pallas import tpu as pltpu

_MEMORY_SIZE = 65536
_KEY_DIM = 64
_VALUE_DIM = 64
_BQ = 1024
_BM = 2048
_NUM_M_BLOCKS = _MEMORY_SIZE // _BM
_SCALE = 1.4426950408889634 / (_KEY_DIM ** 0.5)


def _attn_kernel(q_ref, k_ref, v_ref, wq_ref, bq_ref, wv_ref, bv_ref,
                 o_ref, q_scratch, qn_scratch, acc_scratch, m_scratch):
    i = pl.program_id(0)

    @pl.when(i == 0)
    def _init():
        q = jax.lax.dot_general(q_ref[...], wq_ref[...],
                                (((1,), (0,)), ((), ())),
                                preferred_element_type=jnp.float32)
        q_b = ((q + bq_ref[...]) * _SCALE).astype(jnp.bfloat16)
        q_scratch[...] = q_b
        q32 = q_b.astype(jnp.float32)
        qn = jnp.sqrt(jnp.sum(q32 * q32, axis=1, keepdims=True))
        qn_scratch[...] = jnp.broadcast_to(qn, qn_scratch.shape)
        m_scratch[...] = jnp.full_like(m_scratch, -jnp.inf)
        acc_scratch[...] = jnp.zeros_like(acc_scratch)

    k32 = k_ref[...]
    s = jax.lax.dot_general(q_scratch[...], k32.astype(jnp.bfloat16),
                            (((1,), (1,)), ((), ())),
                            preferred_element_type=jnp.float32)
    ksq = jnp.sum(k32 * k32, axis=1, keepdims=True)
    kmax = jnp.sqrt(jnp.max(ksq)) * 1.01
    m_prev = m_scratch[...]
    m_next = jnp.maximum(m_prev, qn_scratch[...] * kmax)
    alpha = jnp.exp2(m_prev - m_next)
    p_b = jnp.exp2(s - m_next[:, :1]).astype(jnp.bfloat16)
    v_aug = jnp.concatenate(
        [v_ref[...].astype(jnp.bfloat16),
         jnp.ones((_BM, 128 - _VALUE_DIM), dtype=jnp.bfloat16)], axis=1)
    pv = jax.lax.dot_general(p_b, v_aug, (((1,), (0,)), ((), ())),
                             preferred_element_type=jnp.float32)
    acc_scratch[...] = acc_scratch[...] * alpha[:, :1] + pv
    m_scratch[...] = m_next

    @pl.when(i == _NUM_M_BLOCKS - 1)
    def _fin():
        read = (acc_scratch[:, :_VALUE_DIM]
                / acc_scratch[:, _VALUE_DIM:_VALUE_DIM + 1])
        out = jax.lax.dot_general(read, wv_ref[...], (((1,), (0,)), ((), ())),
                                  preferred_element_type=jnp.float32)
        o_ref[...] = out + bv_ref[...]


def _attention(q2d, mem_keys, mem_values, Wq, bq2, Wv, bv2, interpret=False):
    return pl.pallas_call(
        _attn_kernel,
        grid=(_NUM_M_BLOCKS,),
        in_specs=[
            pl.BlockSpec((_BQ, _KEY_DIM), lambda i: (0, 0)),
            pl.BlockSpec((_BM, _KEY_DIM), lambda i: (i, 0)),
            pl.BlockSpec((_BM, _VALUE_DIM), lambda i: (i, 0)),
            pl.BlockSpec((_KEY_DIM, _KEY_DIM), lambda i: (0, 0)),
            pl.BlockSpec((1, _KEY_DIM), lambda i: (0, 0)),
            pl.BlockSpec((_VALUE_DIM, _VALUE_DIM), lambda i: (0, 0)),
            pl.BlockSpec((1, _VALUE_DIM), lambda i: (0, 0)),
        ],
        out_specs=pl.BlockSpec((_BQ, _VALUE_DIM), lambda i: (0, 0)),
        out_shape=jax.ShapeDtypeStruct((_BQ, _VALUE_DIM), jnp.float32),
        scratch_shapes=[
            pltpu.VMEM((_BQ, _KEY_DIM), jnp.bfloat16),
            pltpu.VMEM((_BQ, 128), jnp.float32),
            pltpu.VMEM((_BQ, 128), jnp.float32),
            pltpu.VMEM((_BQ, 128), jnp.float32),
        ],
        compiler_params=pltpu.CompilerParams(
            dimension_semantics=("arbitrary",)),
        interpret=interpret,
    )(q2d, mem_keys, mem_values, Wq, bq2, Wv, bv2)


def kernel(queries, mem_keys, mem_values, Wq, bq, Wv, bv):
    b, n, _ = queries.shape
    q2d = queries.reshape(b * n, _KEY_DIM)
    out = _attention(q2d, mem_keys, mem_values,
                     Wq, bq.reshape(1, -1), Wv, bv.reshape(1, -1))
    return out.reshape(b, n, _VALUE_DIM)
